# Initial kernel scaffold; baseline (speedup 1.0000x reference)
import jax
import jax.numpy as jnp
from jax.experimental import pallas as pl


def kernel(x, batch, W1, b1, prelu_a, W2, b2):
    raise NotImplementedError("write your pallas kernel here")


# speedup vs baseline: 3.9517x; 3.9517x over previous
import functools

import jax
import jax.numpy as jnp
from jax import lax
from jax.experimental import pallas as pl
from jax.experimental.pallas import tpu as pltpu
from jax.experimental.pallas import tpu_sc as plsc

N_NODES = 100000
D = 128
NUM_GRAPHS = 512

NC = 2
NS = 16
NW = NC * NS

CHUNK = 80
NCHUNKS = N_NODES // CHUNK
ROWS = 640
RPT = ROWS // NS
CW = 128


def _sc_segment_sum(x3d, batch2d, zsum, zcnt, ones):
    mesh = plsc.VectorSubcoreMesh(core_axis_name="c", subcore_axis_name="s")

    @functools.partial(
        pl.kernel,
        out_type=[
            jax.ShapeDtypeStruct((NC, ROWS, D), jnp.float32),
            jax.ShapeDtypeStruct((NC, ROWS, CW), jnp.float32),
        ],
        mesh=mesh,
        scratch_types=[
            pltpu.VMEM((CHUNK,), jnp.int32),
            pltpu.VMEM((CHUNK, D), jnp.float32),
            pltpu.VMEM((CHUNK, CW), jnp.float32),
            pltpu.VMEM_SHARED((ROWS, D), jnp.float32),
            pltpu.VMEM_SHARED((ROWS, CW), jnp.float32),
        ],
    )
    def seg_sum(x_hbm, b_hbm, zsum_hbm, zcnt_hbm, ones_hbm,
                sums_out, cnts_out, idx_v, row_v, ones_v, ssum, scnt):
        cid = lax.axis_index("c")
        sid = lax.axis_index("s")
        wid = sid * NC + cid

        pltpu.sync_copy(zsum_hbm.at[pl.ds(RPT * sid, RPT)],
                        ssum.at[pl.ds(RPT * sid, RPT)])
        pltpu.sync_copy(zcnt_hbm.at[pl.ds(RPT * sid, RPT)],
                        scnt.at[pl.ds(RPT * sid, RPT)])
        pltpu.sync_copy(ones_hbm, ones_v)
        plsc.subcore_barrier()

        nc_mine = jnp.where(wid < NCHUNKS % NW, NCHUNKS // NW + 1, NCHUNKS // NW)

        def body(i, carry):
            c = wid + NW * i
            pltpu.sync_copy(b_hbm.at[c], idx_v)
            pltpu.sync_copy(x_hbm.at[c], row_v)
            pltpu.sync_copy(row_v, ssum.at[idx_v], add=True)
            pltpu.sync_copy(ones_v, scnt.at[idx_v], add=True)
            return carry

        lax.fori_loop(0, nc_mine, body, 0)
        plsc.subcore_barrier()

        pltpu.sync_copy(ssum.at[pl.ds(RPT * sid, RPT)],
                        sums_out.at[cid, pl.ds(RPT * sid, RPT)])
        pltpu.sync_copy(scnt.at[pl.ds(RPT * sid, RPT)],
                        cnts_out.at[cid, pl.ds(RPT * sid, RPT)])

    return seg_sum(x3d, batch2d, zsum, zcnt, ones)


def _mlp_body(sums_ref, cnts_ref, w1_ref, b1_ref, a_ref, w2_ref, b2_ref, out_ref):
    s = sums_ref[0] + sums_ref[1]
    cnt = cnts_ref[0] + cnts_ref[1]
    emb = s[:NUM_GRAPHS] / jnp.clip(cnt[:NUM_GRAPHS, 0:1], 1.0, None)
    h = jnp.dot(emb, w1_ref[:], preferred_element_type=jnp.float32) + b1_ref[:]
    a = a_ref[0, 0]
    h = jnp.where(h >= 0, h, a * h)
    out_ref[:] = (
        jnp.dot(h, w2_ref[:], preferred_element_type=jnp.float32) + b2_ref[:]
    )


def kernel(x, batch, W1, b1, prelu_a, W2, b2):
    x3d = x.reshape(NCHUNKS, CHUNK, D)
    batch2d = batch.astype(jnp.int32).reshape(NCHUNKS, CHUNK)
    zsum = jnp.zeros((ROWS, D), jnp.float32)
    zcnt = jnp.zeros((ROWS, CW), jnp.float32)
    ones = jnp.ones((CHUNK, CW), jnp.float32)

    sums, cnts = _sc_segment_sum(x3d, batch2d, zsum, zcnt, ones)

    return pl.pallas_call(
        _mlp_body,
        out_shape=jax.ShapeDtypeStruct((NUM_GRAPHS, D), jnp.float32),
    )(sums, cnts, W1, b1.reshape(1, D), prelu_a.reshape(1, 1),
      W2, b2.reshape(1, D))


# speedup vs baseline: 6.5532x; 1.6583x over previous
import functools

import jax
import jax.numpy as jnp
from jax import lax
from jax.experimental import pallas as pl
from jax.experimental.pallas import tpu as pltpu
from jax.experimental.pallas import tpu_sc as plsc

N_NODES = 100000
D = 128
NUM_GRAPHS = 512

NC = 2
NS = 16
NW = NC * NS

CHUNK = 80
NCHUNKS = N_NODES // CHUNK
ROWS = 640
RPT = ROWS // NS
CW = 128


def _sc_segment_sum(x3d, batch2d, zsum, zcnt, ones):
    mesh = plsc.VectorSubcoreMesh(core_axis_name="c", subcore_axis_name="s")

    @functools.partial(
        pl.kernel,
        out_type=[
            jax.ShapeDtypeStruct((NC, ROWS, D), jnp.float32),
            jax.ShapeDtypeStruct((NC, ROWS, CW), jnp.float32),
        ],
        mesh=mesh,
        scratch_types=[
            pltpu.VMEM((CHUNK,), jnp.int32),
            pltpu.VMEM((CHUNK,), jnp.int32),
            pltpu.VMEM((CHUNK, D), jnp.float32),
            pltpu.VMEM((CHUNK, D), jnp.float32),
            pltpu.VMEM((CHUNK, CW), jnp.float32),
            pltpu.VMEM_SHARED((ROWS, D), jnp.float32),
            pltpu.VMEM_SHARED((ROWS, CW), jnp.float32),
            pltpu.SemaphoreType.DMA,
            pltpu.SemaphoreType.DMA,
        ],
    )
    def seg_sum(x_hbm, b_hbm, zsum_hbm, zcnt_hbm, ones_hbm,
                sums_out, cnts_out, idx_v0, idx_v1, row_v0, row_v1, ones_v,
                ssum, scnt, sem0, sem1):
        cid = lax.axis_index("c")
        sid = lax.axis_index("s")
        wid = sid * NC + cid

        pltpu.sync_copy(zsum_hbm.at[pl.ds(RPT * sid, RPT)],
                        ssum.at[pl.ds(RPT * sid, RPT)])
        pltpu.sync_copy(zcnt_hbm.at[pl.ds(RPT * sid, RPT)],
                        scnt.at[pl.ds(RPT * sid, RPT)])
        pltpu.sync_copy(ones_hbm, ones_v)
        plsc.subcore_barrier()

        nc_mine = jnp.where(wid < NCHUNKS % NW, NCHUNKS // NW + 1, NCHUNKS // NW)

        def do_slot(j, idxb, rowb, semb):
            c = wid + NW * j
            pltpu.make_async_copy(b_hbm.at[c], idxb, semb).wait()
            pltpu.make_async_copy(x_hbm.at[c], rowb, semb).wait()
            pltpu.sync_copy(rowb, ssum.at[idxb], add=True)
            pltpu.sync_copy(ones_v, scnt.at[idxb], add=True)

            @pl.when(j + 2 < nc_mine)
            def _():
                c2 = wid + NW * (j + 2)
                pltpu.async_copy(b_hbm.at[c2], idxb, semb)
                pltpu.async_copy(x_hbm.at[c2], rowb, semb)

        pltpu.async_copy(b_hbm.at[wid], idx_v0, sem0)
        pltpu.async_copy(x_hbm.at[wid], row_v0, sem0)
        pltpu.async_copy(b_hbm.at[wid + NW], idx_v1, sem1)
        pltpu.async_copy(x_hbm.at[wid + NW], row_v1, sem1)

        def body(g, carry):
            j0 = 2 * g

            @pl.when(j0 < nc_mine)
            def _():
                do_slot(j0, idx_v0, row_v0, sem0)

            j1 = 2 * g + 1

            @pl.when(j1 < nc_mine)
            def _():
                do_slot(j1, idx_v1, row_v1, sem1)

            return carry

        lax.fori_loop(0, (NCHUNKS // NW + 2) // 2, body, 0)
        plsc.subcore_barrier()

        pltpu.sync_copy(ssum.at[pl.ds(RPT * sid, RPT)],
                        sums_out.at[cid, pl.ds(RPT * sid, RPT)])
        pltpu.sync_copy(scnt.at[pl.ds(RPT * sid, RPT)],
                        cnts_out.at[cid, pl.ds(RPT * sid, RPT)])

    return seg_sum(x3d, batch2d, zsum, zcnt, ones)


def _mlp_body(sums_ref, cnts_ref, w1_ref, b1_ref, a_ref, w2_ref, b2_ref, out_ref):
    s = sums_ref[0] + sums_ref[1]
    cnt = cnts_ref[0] + cnts_ref[1]
    emb = s[:NUM_GRAPHS] / jnp.clip(cnt[:NUM_GRAPHS, 0:1], 1.0, None)
    h = jnp.dot(emb, w1_ref[:], preferred_element_type=jnp.float32) + b1_ref[:]
    a = a_ref[0, 0]
    h = jnp.where(h >= 0, h, a * h)
    out_ref[:] = (
        jnp.dot(h, w2_ref[:], preferred_element_type=jnp.float32) + b2_ref[:]
    )


def kernel(x, batch, W1, b1, prelu_a, W2, b2):
    x3d = x.reshape(NCHUNKS, CHUNK, D)
    batch2d = batch.astype(jnp.int32).reshape(NCHUNKS, CHUNK)
    zsum = jnp.zeros((ROWS, D), jnp.float32)
    zcnt = jnp.zeros((ROWS, CW), jnp.float32)
    ones = jnp.ones((CHUNK, CW), jnp.float32)

    sums, cnts = _sc_segment_sum(x3d, batch2d, zsum, zcnt, ones)

    return pl.pallas_call(
        _mlp_body,
        out_shape=jax.ShapeDtypeStruct((NUM_GRAPHS, D), jnp.float32),
    )(sums, cnts, W1, b1.reshape(1, D), prelu_a.reshape(1, 1),
      W2, b2.reshape(1, D))


# speedup vs baseline: 8.6347x; 1.3176x over previous
import functools

import jax
import jax.numpy as jnp
from jax import lax
from jax.experimental import pallas as pl
from jax.experimental.pallas import tpu as pltpu
from jax.experimental.pallas import tpu_sc as plsc

N_NODES = 100000
D = 128
NUM_GRAPHS = 512

NC = 2
NS = 16
NW = NC * NS

CHUNK = 80
NCHUNKS = N_NODES // CHUNK
ROWS = 640
RPT = ROWS // NS

GH = 32
GL = 16
CNT_BLK = 2048
N_PAD = 102400


def _sc_segment_sum(x3d, batch2d, zsum):
    mesh = plsc.VectorSubcoreMesh(core_axis_name="c", subcore_axis_name="s")

    @functools.partial(
        pl.kernel,
        out_type=jax.ShapeDtypeStruct((NC, ROWS, D), jnp.float32),
        mesh=mesh,
        scratch_types=[
            pltpu.VMEM((CHUNK,), jnp.int32),
            pltpu.VMEM((CHUNK,), jnp.int32),
            pltpu.VMEM((CHUNK, D), jnp.float32),
            pltpu.VMEM((CHUNK, D), jnp.float32),
            pltpu.VMEM_SHARED((ROWS, D), jnp.float32),
            pltpu.SemaphoreType.DMA,
            pltpu.SemaphoreType.DMA,
        ],
    )
    def seg_sum(x_hbm, b_hbm, zsum_hbm, sums_out,
                idx_v0, idx_v1, row_v0, row_v1, ssum, sem0, sem1):
        cid = lax.axis_index("c")
        sid = lax.axis_index("s")
        wid = sid * NC + cid

        pltpu.sync_copy(zsum_hbm.at[pl.ds(RPT * sid, RPT)],
                        ssum.at[pl.ds(RPT * sid, RPT)])
        plsc.subcore_barrier()

        nc_mine = jnp.where(wid < NCHUNKS % NW, NCHUNKS // NW + 1, NCHUNKS // NW)

        def do_slot(j, idxb, rowb, semb):
            c = wid + NW * j
            pltpu.make_async_copy(b_hbm.at[c], idxb, semb).wait()
            pltpu.make_async_copy(x_hbm.at[c], rowb, semb).wait()
            pltpu.sync_copy(rowb, ssum.at[idxb], add=True)

            @pl.when(j + 2 < nc_mine)
            def _():
                c2 = wid + NW * (j + 2)
                pltpu.async_copy(b_hbm.at[c2], idxb, semb)
                pltpu.async_copy(x_hbm.at[c2], rowb, semb)

        pltpu.async_copy(b_hbm.at[wid], idx_v0, sem0)
        pltpu.async_copy(x_hbm.at[wid], row_v0, sem0)
        pltpu.async_copy(b_hbm.at[wid + NW], idx_v1, sem1)
        pltpu.async_copy(x_hbm.at[wid + NW], row_v1, sem1)

        def body(g, carry):
            j0 = 2 * g

            @pl.when(j0 < nc_mine)
            def _():
                do_slot(j0, idx_v0, row_v0, sem0)

            j1 = 2 * g + 1

            @pl.when(j1 < nc_mine)
            def _():
                do_slot(j1, idx_v1, row_v1, sem1)

            return carry

        lax.fori_loop(0, (NCHUNKS // NW + 2) // 2, body, 0)
        plsc.subcore_barrier()

        pltpu.sync_copy(ssum.at[pl.ds(RPT * sid, RPT)],
                        sums_out.at[cid, pl.ds(RPT * sid, RPT)])

    return seg_sum(x3d, batch2d, zsum)


def _counts_body(b_ref, out_ref):
    i = pl.program_id(0)

    @pl.when(i == 0)
    def _():
        out_ref[:] = jnp.zeros_like(out_ref)

    b = b_ref[:]
    hi = b >> 4
    lo = b & 15
    ih = lax.broadcasted_iota(jnp.int32, (GH, 1), 0)
    il = lax.broadcasted_iota(jnp.int32, (GL, 1), 0)
    oh_hi = (hi == ih).astype(jnp.bfloat16)
    oh_lo = (lo == il).astype(jnp.bfloat16)
    out_ref[:] += lax.dot_general(
        oh_hi, oh_lo, (((1,), (1,)), ((), ())),
        preferred_element_type=jnp.float32)


def _tc_counts(brow):
    return pl.pallas_call(
        _counts_body,
        grid=(N_PAD // CNT_BLK,),
        in_specs=[pl.BlockSpec((1, CNT_BLK), lambda i: (0, i))],
        out_specs=pl.BlockSpec((GH, GL), lambda i: (0, 0)),
        out_shape=jax.ShapeDtypeStruct((GH, GL), jnp.float32),
    )(brow)


def _mlp_body(sums_ref, cnt_ref, w1_ref, b1_ref, a_ref, w2_ref, b2_ref, out_ref):
    s = sums_ref[0] + sums_ref[1]
    emb = s[:NUM_GRAPHS] / jnp.clip(cnt_ref[:], 1.0, None)
    h = jnp.dot(emb, w1_ref[:], preferred_element_type=jnp.float32) + b1_ref[:]
    a = a_ref[0, 0]
    h = jnp.where(h >= 0, h, a * h)
    out_ref[:] = (
        jnp.dot(h, w2_ref[:], preferred_element_type=jnp.float32) + b2_ref[:]
    )


def kernel(x, batch, W1, b1, prelu_a, W2, b2):
    batch32 = batch.astype(jnp.int32)
    x3d = x.reshape(NCHUNKS, CHUNK, D)
    batch2d = batch32.reshape(NCHUNKS, CHUNK)
    zsum = jnp.zeros((ROWS, D), jnp.float32)
    brow = jnp.concatenate(
        [batch32, jnp.full((N_PAD - N_NODES,), 1 << 20, jnp.int32)]
    ).reshape(1, N_PAD)

    cnts = _tc_counts(brow)
    sums = _sc_segment_sum(x3d, batch2d, zsum)

    return pl.pallas_call(
        _mlp_body,
        out_shape=jax.ShapeDtypeStruct((NUM_GRAPHS, D), jnp.float32),
    )(sums, cnts.reshape(NUM_GRAPHS, 1), W1, b1.reshape(1, D),
      prelu_a.reshape(1, 1), W2, b2.reshape(1, D))
